# baseline (speedup 1.0000x reference)
import jax
import jax.numpy as jnp
from jax.experimental import pallas as pl
from jax.experimental.pallas import tpu as pltpu

_INPUT_DIM = 2048
_NUM_CODES = 1024
_CODE_DIM = 256
_GROUPS = 2
_T_TILE = 512

_MM_PREC = jax.lax.Precision.DEFAULT


def _vq_kernel(x_ref, w_ref, b_ref, cbt_ref, q_ref, counts_ref, psum_ref,
               l_scr):
    s = pl.program_id(0)
    last = pl.num_programs(0) - 1

    @pl.when(s == 0)
    def _init():
        counts_ref[...] = jnp.zeros_like(counts_ref)
        psum_ref[...] = jnp.zeros_like(psum_ref)

    @pl.when(s < last)
    def _matmul():
        logits = jax.lax.dot_general(
            w_ref[...], x_ref[0], (((1,), (0,)), ((), ())),
            precision=_MM_PREC, preferred_element_type=jnp.float32)
        l_scr[s % 2] = logits + b_ref[...]

    @pl.when(s > 0)
    def _epilogue():
        l3 = l_scr[(s - 1) % 2].reshape(_GROUPS, _NUM_CODES, _T_TILE)
        m = jnp.max(l3, axis=1)
        iota = jax.lax.broadcasted_iota(jnp.int32, l3.shape, 1)
        k = jnp.min(jnp.where(l3 == m[:, None, :], iota, _NUM_CODES), axis=1)
        onehot = (iota == k[:, None, :]).astype(jnp.float32)

        counts_ref[...] += jnp.sum(onehot, axis=2)

        p = jnp.exp(l3 - m[:, None, :])
        rinv = 1.0 / jnp.sum(p, axis=1)
        psum_ref[...] += jnp.sum(p * rinv[:, None, :], axis=2)

        for g in range(_GROUPS):
            qg = jax.lax.dot_general(
                cbt_ref[g], onehot[g], (((1,), (0,)), ((), ())),
                precision=_MM_PREC, preferred_element_type=jnp.float32)
            q_ref[0, g * _CODE_DIM:(g + 1) * _CODE_DIM, :] = qg


def kernel(x, W, b, codebook):
    bsz, fsz, tsz = x.shape
    gv = _GROUPS * _NUM_CODES
    n_tok = bsz * tsz
    cbt = jnp.transpose(codebook[0], (0, 2, 1))
    b2 = b.reshape(gv, 1)

    tt = tsz // _T_TILE
    nsteps = bsz * tt + 1

    def _x_idx(s):
        c = jnp.minimum(s, bsz * tt - 1)
        return (c // tt, 0, c % tt)

    def _q_idx(s):
        e = jnp.maximum(s, 1) - 1
        return (e // tt, 0, e % tt)

    q, counts, psum = pl.pallas_call(
        _vq_kernel,
        grid=(nsteps,),
        in_specs=[
            pl.BlockSpec((1, fsz, _T_TILE), _x_idx),
            pl.BlockSpec((gv, fsz), lambda s: (0, 0)),
            pl.BlockSpec((gv, 1), lambda s: (0, 0)),
            pl.BlockSpec((_GROUPS, _CODE_DIM, _NUM_CODES), lambda s: (0, 0, 0)),
        ],
        out_specs=[
            pl.BlockSpec((1, _GROUPS * _CODE_DIM, _T_TILE), _q_idx),
            pl.BlockSpec((_GROUPS, _NUM_CODES), lambda s: (0, 0)),
            pl.BlockSpec((_GROUPS, _NUM_CODES), lambda s: (0, 0)),
        ],
        out_shape=[
            jax.ShapeDtypeStruct((bsz, _GROUPS * _CODE_DIM, tsz), jnp.float32),
            jax.ShapeDtypeStruct((_GROUPS, _NUM_CODES), jnp.float32),
            jax.ShapeDtypeStruct((_GROUPS, _NUM_CODES), jnp.float32),
        ],
        scratch_shapes=[pltpu.VMEM((2, gv, _T_TILE), jnp.float32)],
        compiler_params=pltpu.CompilerParams(
            dimension_semantics=("arbitrary",),
        ),
    )(x, W, b2, cbt)

    hard_probs = counts / n_tok
    code_perplexity = jnp.sum(
        jnp.exp(-jnp.sum(hard_probs * jnp.log(hard_probs + 1e-7), axis=-1)))
    avg_probs = psum / n_tok
    prob_perplexity = jnp.sum(
        jnp.exp(-jnp.sum(avg_probs * jnp.log(avg_probs + 1e-7), axis=-1)))
    num_vars = _NUM_CODES * _GROUPS
    diversity = (num_vars - prob_perplexity) / num_vars
    return q, diversity, code_perplexity, prob_perplexity
